# baseline (speedup 1.0000x reference)
import functools

import jax
import jax.numpy as jnp
from jax import lax
from jax.experimental import pallas as pl
from jax.experimental.pallas import tpu as pltpu
from jax.experimental.pallas import tpu_sc as plsc

NC = 2
NS = 16
NW = NC * NS
L = 16

TCOL = 128
WIN_TC = 5
WIN_C = WIN_TC * TCOL
TAIL_W = 320


def kernel(anime_id, embedding_table):
    B = anime_id.shape[0]
    V, D = embedding_table.shape
    idx = anime_id.astype(jnp.int32)
    table_t = embedding_table.T

    n_tc = -(-V // TCOL)
    tc_per_w = -(-n_tc // NW)
    full_wins = tc_per_w // WIN_TC
    last_own = n_tc - (NW - 1) * tc_per_w
    last_full = last_own // WIN_TC

    OUT_ROWS = B + 128
    n_grp = B // L
    b_chunk = B // NW

    mesh = plsc.VectorSubcoreMesh(core_axis_name="c", subcore_axis_name="s")

    @functools.partial(
        pl.kernel,
        mesh=mesh,
        out_type=jax.ShapeDtypeStruct((OUT_ROWS, 128), jnp.float32),
        compiler_params=pltpu.CompilerParams(needs_layout_passes=False),
        scratch_types=[
            pltpu.VMEM((B,), jnp.int32),
            pltpu.VMEM((B + L,), jnp.int32),
            pltpu.VMEM((B + L,), jnp.int32),
            pltpu.VMEM((2, 4 * WIN_TC, 8, TCOL), jnp.float32),
            pltpu.VMEM((L, 128), jnp.float32),
            pltpu.SemaphoreType.DMA,
            pltpu.SemaphoreType.DMA,
            pltpu.SemaphoreType.DMA,
        ],
    )
    def gather_kernel(idx_hbm, table_hbm, out_hbm, idx_all, fid, fpos,
                      win, stage, sem, sem_w0, sem_w1):
        wid = lax.axis_index("s") * NC + lax.axis_index("c")

        def bcast(g, carry):
            off = ((wid + g) % NW) * b_chunk
            pltpu.async_copy(
                idx_hbm.at[pl.ds(off, b_chunk)],
                idx_all.at[pl.ds(off, b_chunk)],
                sem,
            )
            return carry

        lax.fori_loop(0, NW, bcast, 0)

        def bdrain(g, carry):
            pltpu.make_async_copy(
                idx_hbm.at[pl.ds(0, b_chunk)],
                idx_all.at[pl.ds(0, b_chunk)],
                sem,
            ).wait()
            return carry

        lax.fori_loop(0, NW, bdrain, 0)

        lo = wid * (tc_per_w * TCOL)
        hi = jnp.minimum(lo + tc_per_w * TCOL, V)

        def filt(g, cnt):
            vec = idx_all[pl.ds(g * L, L)]
            posv = lax.iota(jnp.int32, L) + g * L
            m = (vec >= lo) & (vec < hi)
            cs = plsc.cumsum(m.astype(jnp.int32))
            dstp = jnp.where(m, cnt + cs - 1, B)
            plsc.store_scatter(fid, [dstp], vec)
            plsc.store_scatter(fpos, [dstp], posv)
            return cnt + cs[L - 1]

        cnt = lax.fori_loop(0, n_grp, filt, jnp.int32(0))
        n_g = (cnt + L - 1) // L

        kv0 = lax.iota(jnp.int32, L)
        kv1 = kv0 + L

        def do_window(wbuf, w_base_c, width, ntc):
            tv0 = (kv0 // 8) * ntc
            tv1 = tv0 + 2 * ntc
            sv = kv0 % 8

            def scan(g, carry):
                vec = fid[pl.ds(g * L, L)]
                posv = fpos[pl.ds(g * L, L)]
                valid = (lax.iota(jnp.int32, L) + g * L) < cnt
                m = valid & (vec >= w_base_c) & (vec < w_base_c + width)
                nm = plsc.all_reduce_population_count(m)

                @pl.when(nm[0] > 0)
                def _():
                    cl = jnp.clip(vec - w_base_c, 0, width - 1)
                    clj = cl // TCOL
                    cll = cl - clj * TCOL
                    for l in range(L):
                        jv = jnp.full((L,), clj[l], jnp.int32)
                        cv = jnp.full((L,), cll[l], jnp.int32)
                        r0 = plsc.load_gather(wbuf, [tv0 + jv, sv, cv])
                        r1 = plsc.load_gather(wbuf, [tv1 + jv, sv, cv])
                        stage[l, pl.ds(0, L)] = r0
                        stage[l, pl.ds(L, L)] = r1
                    dummy = B + (wid % 8) * L + lax.iota(jnp.int32, L)
                    pos_sel = jnp.where(m, posv, dummy)
                    pltpu.async_copy(stage, out_hbm.at[pos_sel], sem)
                    pltpu.make_async_copy(
                        stage, out_hbm.at[pl.ds(0, L)], sem
                    ).wait()

                return carry

            lax.fori_loop(0, n_g, scan, 0)

        n_full = jnp.where(wid < NW - 1, full_wins, last_full)

        def fire_tiles(base_c, buf, sem_w, ntc):
            for k1 in range(4):
                for j in range(ntc):
                    pltpu.async_copy(
                        table_hbm.at[
                            pl.ds(8 * k1, 8),
                            pl.ds(pl.multiple_of(base_c + j * TCOL, TCOL), TCOL),
                        ],
                        win.at[buf].at[k1 * ntc + j],
                        sem_w,
                    )

        def drain_tiles(buf, sem_w, ntc):
            for t in range(4 * ntc):
                pltpu.make_async_copy(
                    table_hbm.at[pl.ds(0, 8), pl.ds(0, TCOL)],
                    win.at[buf].at[t],
                    sem_w,
                ).wait()

        def issue_win(g, buf, sem_w):
            @pl.when(g < n_full)
            def _():
                fire_tiles(lo + g * WIN_C, buf, sem_w, WIN_TC)

        def wait_win(buf, sem_w):
            drain_tiles(buf, sem_w, WIN_TC)

        issue_win(jnp.int32(0), 0, sem_w0)

        def wpair(p, carry):
            g0 = p * 2
            g1 = g0 + 1
            issue_win(g1, 1, sem_w1)

            @pl.when(g0 < n_full)
            def _():
                wait_win(0, sem_w0)
                do_window(win.at[0], lo + g0 * WIN_C, WIN_C, WIN_TC)

            issue_win(g1 + 1, 0, sem_w0)

            @pl.when(g1 < n_full)
            def _():
                wait_win(1, sem_w1)
                do_window(win.at[1], lo + g1 * WIN_C, WIN_C, WIN_TC)

            return carry

        lax.fori_loop(0, (full_wins + 1) // 2, wpair, 0)

        t1 = jnp.where(wid == NW - 1, hi - 320, hi - 256)
        fire_tiles(t1, 0, sem_w0, 2)
        drain_tiles(0, sem_w0, 2)
        do_window(win.at[0], t1, 256, 2)
        t2 = jnp.where(wid == NW - 1, hi - 64, hi - 128)
        fire_tiles(t2, 1, sem_w1, 1)
        drain_tiles(1, sem_w1, 1)
        do_window(win.at[1], t2, 128, 1)

        drainv = B + (wid % 8) * L + lax.iota(jnp.int32, L)
        pltpu.async_copy(out_hbm.at[drainv], stage, sem)
        pltpu.make_async_copy(out_hbm.at[pl.ds(0, L)], stage, sem).wait()

    out = gather_kernel(idx, table_t)
    return out[:B, :D]


# speedup vs baseline: 1.1795x; 1.1795x over previous
import functools

import jax
import jax.numpy as jnp
from jax import lax
from jax.experimental import pallas as pl
from jax.experimental.pallas import tpu as pltpu
from jax.experimental.pallas import tpu_sc as plsc

NC = 2
NS = 16
NW = NC * NS
L = 16

TCOL = 128
WIN_TC = 9
WIN_C = WIN_TC * TCOL
TAIL_W = 320


def kernel(anime_id, embedding_table):
    B = anime_id.shape[0]
    V, D = embedding_table.shape
    idx = anime_id.astype(jnp.int32)
    table_t = embedding_table.T

    n_tc = -(-V // TCOL)
    tc_per_w = -(-n_tc // NW)
    full_wins = tc_per_w // WIN_TC
    last_own = n_tc - (NW - 1) * tc_per_w
    last_full = last_own // WIN_TC

    OUT_ROWS = B + 128
    n_grp = B // L
    b_chunk = B // NW

    mesh = plsc.VectorSubcoreMesh(core_axis_name="c", subcore_axis_name="s")

    @functools.partial(
        pl.kernel,
        mesh=mesh,
        out_type=jax.ShapeDtypeStruct((OUT_ROWS, 128), jnp.float32),
        compiler_params=pltpu.CompilerParams(needs_layout_passes=False),
        scratch_types=[
            pltpu.VMEM((B,), jnp.int32),
            pltpu.VMEM((B + L,), jnp.int32),
            pltpu.VMEM((B + L,), jnp.int32),
            pltpu.VMEM((2, 4, 8, WIN_C), jnp.float32),
            pltpu.VMEM((L, 128), jnp.float32),
            pltpu.SemaphoreType.DMA,
            pltpu.SemaphoreType.DMA,
            pltpu.SemaphoreType.DMA,
        ],
    )
    def gather_kernel(idx_hbm, table_hbm, out_hbm, idx_all, fid, fpos,
                      win, stage, sem, sem_w0, sem_w1):
        wid = lax.axis_index("s") * NC + lax.axis_index("c")

        def bcast(g, carry):
            off = ((wid + g) % NW) * b_chunk
            pltpu.async_copy(
                idx_hbm.at[pl.ds(off, b_chunk)],
                idx_all.at[pl.ds(off, b_chunk)],
                sem,
            )
            return carry

        lax.fori_loop(0, NW, bcast, 0)

        def bdrain(g, carry):
            pltpu.make_async_copy(
                idx_hbm.at[pl.ds(0, b_chunk)],
                idx_all.at[pl.ds(0, b_chunk)],
                sem,
            ).wait()
            return carry

        lax.fori_loop(0, NW, bdrain, 0)

        lo = wid * (tc_per_w * TCOL)
        hi = jnp.minimum(lo + tc_per_w * TCOL, V)

        def filt(g, cnt):
            vec = idx_all[pl.ds(g * L, L)]
            posv = lax.iota(jnp.int32, L) + g * L
            m = (vec >= lo) & (vec < hi)
            cs = plsc.cumsum(m.astype(jnp.int32))
            dstp = jnp.where(m, cnt + cs - 1, B)
            plsc.store_scatter(fid, [dstp], vec)
            plsc.store_scatter(fpos, [dstp], posv)
            return cnt + cs[L - 1]

        cnt = lax.fori_loop(0, n_grp, filt, jnp.int32(0))
        n_g = (cnt + L - 1) // L

        kv0 = lax.iota(jnp.int32, L)
        kv1 = kv0 + L

        tv0 = kv0 // 8
        tv1 = tv0 + 2
        sv = kv0 % 8

        def do_window(wbuf, w_base_c, width, ntc):
            def scan(g, carry):
                vec = fid[pl.ds(g * L, L)]
                posv = fpos[pl.ds(g * L, L)]
                valid = (lax.iota(jnp.int32, L) + g * L) < cnt
                m = valid & (vec >= w_base_c) & (vec < w_base_c + width)
                nm = plsc.all_reduce_population_count(m)

                @pl.when(nm[0] > 0)
                def _():
                    cl = jnp.clip(vec - w_base_c, 0, width - 1)
                    for l in range(L):
                        cv = jnp.full((L,), cl[l], jnp.int32)
                        r0 = plsc.load_gather(wbuf, [tv0, sv, cv])
                        r1 = plsc.load_gather(wbuf, [tv1, sv, cv])
                        stage[l, pl.ds(0, L)] = r0
                        stage[l, pl.ds(L, L)] = r1
                    dummy = B + (wid % 8) * L + lax.iota(jnp.int32, L)
                    pos_sel = jnp.where(m, posv, dummy)
                    pltpu.async_copy(stage, out_hbm.at[pos_sel], sem)
                    pltpu.make_async_copy(
                        stage, out_hbm.at[pl.ds(0, L)], sem
                    ).wait()

                return carry

            lax.fori_loop(0, n_g, scan, 0)

        n_full = jnp.where(wid < NW - 1, full_wins, last_full)

        def fire_tiles(base_c, buf, sem_w, ntc):
            for k1 in range(4):
                pltpu.async_copy(
                    table_hbm.at[
                        pl.ds(8 * k1, 8),
                        pl.ds(pl.multiple_of(base_c, TCOL), ntc * TCOL),
                    ],
                    win.at[buf].at[k1].at[:, pl.ds(0, ntc * TCOL)],
                    sem_w,
                )

        def drain_tiles(buf, sem_w, ntc):
            for k1 in range(4):
                pltpu.make_async_copy(
                    table_hbm.at[pl.ds(0, 8), pl.ds(0, ntc * TCOL)],
                    win.at[buf].at[k1].at[:, pl.ds(0, ntc * TCOL)],
                    sem_w,
                ).wait()

        def issue_win(g, buf, sem_w):
            @pl.when(g < n_full)
            def _():
                fire_tiles(lo + g * WIN_C, buf, sem_w, WIN_TC)

        def wait_win(buf, sem_w):
            drain_tiles(buf, sem_w, WIN_TC)

        issue_win(jnp.int32(0), 0, sem_w0)

        def wpair(p, carry):
            g0 = p * 2
            g1 = g0 + 1
            issue_win(g1, 1, sem_w1)

            @pl.when(g0 < n_full)
            def _():
                wait_win(0, sem_w0)
                do_window(win.at[0], lo + g0 * WIN_C, WIN_C, WIN_TC)

            issue_win(g1 + 1, 0, sem_w0)

            @pl.when(g1 < n_full)
            def _():
                wait_win(1, sem_w1)
                do_window(win.at[1], lo + g1 * WIN_C, WIN_C, WIN_TC)

            return carry

        lax.fori_loop(0, (full_wins + 1) // 2, wpair, 0)

        t1 = jnp.where(wid == NW - 1, hi - 320, hi - 256)
        fire_tiles(t1, 0, sem_w0, 2)
        drain_tiles(0, sem_w0, 2)
        do_window(win.at[0], t1, 256, 2)
        t2 = jnp.where(wid == NW - 1, hi - 64, hi - 128)
        fire_tiles(t2, 1, sem_w1, 1)
        drain_tiles(1, sem_w1, 1)
        do_window(win.at[1], t2, 128, 1)

        drainv = B + (wid % 8) * L + lax.iota(jnp.int32, L)
        pltpu.async_copy(out_hbm.at[drainv], stage, sem)
        pltpu.make_async_copy(out_hbm.at[pl.ds(0, L)], stage, sem).wait()

    out = gather_kernel(idx, table_t)
    return out[:B, :D]
